# Initial kernel scaffold; baseline (speedup 1.0000x reference)
import jax
import jax.numpy as jnp
from jax.experimental import pallas as pl


def kernel(x, edge_index, edge_label_index, W_in, b_in, W_c0, b_c0, W_c1, b_c1, g0, be0, g1, be1, W_out, b_out, W_m1, b_m1, W_m2, b_m2, W_m3, b_m3):
    raise NotImplementedError("write your pallas kernel here")


# speedup vs baseline: 1.8987x; 1.8987x over previous
import functools

import jax
import jax.numpy as jnp
from jax import lax
from jax.experimental import pallas as pl
from jax.experimental.pallas import tpu as pltpu
from jax.experimental.pallas import tpu_sc as plsc

NC = 2
NS = 16
LANES = 16

_MESH = plsc.VectorSubcoreMesh(
    core_axis_name="c", subcore_axis_name="s", num_cores=NC, num_subcores=NS)


def _zero_vmem(buf, rows):
    z = jnp.zeros((LANES,), jnp.float32)

    def body(i, _):
        r = i // 8
        k = i % 8
        buf[r, pl.ds(k * LANES, LANES)] = z
        return 0

    lax.fori_loop(0, rows * 8, body, 0)


def _sc_agg_call(col2d, row2d, h_lo, h_hi, n_pad, want_deg):
    kbt = col2d.shape[0] // NS
    rpt = n_pad // NS
    zrows = 8
    chunk = 32
    assert kbt % chunk == 0

    out_type = [
        jax.ShapeDtypeStruct((n_pad, 128), jnp.float32),
        jax.ShapeDtypeStruct((n_pad, 128), jnp.float32),
    ]
    if want_deg:
        out_type.append(jax.ShapeDtypeStruct((n_pad,), jnp.float32))

    scratch = [
        pltpu.VMEM((chunk, 128), jnp.int32),
        pltpu.VMEM((chunk, 128), jnp.int32),
        pltpu.VMEM((128, 128), jnp.float32),
        pltpu.VMEM((zrows, 128), jnp.float32),
        pltpu.VMEM((128,), jnp.float32),
        pltpu.VMEM_SHARED((n_pad, 128), jnp.float32),
        pltpu.VMEM_SHARED((n_pad,), jnp.float32),
        pltpu.SemaphoreType.DMA,
    ]

    def body(col_hbm, row_hbm, hlo_hbm, hhi_hbm, *rest):
        if want_deg:
            (alo_hbm, ahi_hbm, deg_hbm, colb, rowb, rows_v, zbuf, onesb,
             acc_s, deg_s, sem) = rest
        else:
            (alo_hbm, ahi_hbm, colb, rowb, rows_v, zbuf, onesb,
             acc_s, deg_s, sem) = rest
        c = lax.axis_index("c")
        s = lax.axis_index("s")

        _zero_vmem(zbuf, zrows)
        one = jnp.ones((LANES,), jnp.float32)
        for k in range(8):
            onesb[pl.ds(k * LANES, LANES)] = one

        def zc(k, _):
            pltpu.sync_copy(zbuf, acc_s.at[pl.ds(s * rpt + k * zrows, zrows)])
            return 0
        lax.fori_loop(0, rpt // zrows, zc, 0)

        def zd(k, _):
            pltpu.sync_copy(zbuf.at[0], deg_s.at[pl.ds(s * rpt + k * 128, 128)])
            return 0
        lax.fori_loop(0, rpt // 128, zd, 0)
        plsc.subcore_barrier()

        def make_loop(h_hbm, with_deg):
            def cb(q, _):
                blk0 = s * kbt + q * chunk
                pltpu.sync_copy(col_hbm.at[pl.ds(blk0, chunk)], colb)
                pltpu.sync_copy(row_hbm.at[pl.ds(blk0, chunk)], rowb)

                def eb(j, _):
                    pltpu.async_copy(h_hbm.at[colb.at[j]], rows_v, sem).wait()
                    pltpu.sync_copy(rows_v, acc_s.at[rowb.at[j]], add=True)
                    if with_deg:
                        pltpu.sync_copy(onesb, deg_s.at[rowb.at[j]],
                                        add=True)
                    return 0
                lax.fori_loop(0, chunk, eb, 0)
                return 0
            return cb

        @pl.when(c == 0)
        def _():
            lax.fori_loop(0, kbt // chunk, make_loop(hlo_hbm, want_deg), 0)

        @pl.when(c == 1)
        def _():
            lax.fori_loop(0, kbt // chunk, make_loop(hhi_hbm, False), 0)

        plsc.subcore_barrier()

        @pl.when(c == 0)
        def _():
            pltpu.sync_copy(acc_s.at[pl.ds(s * rpt, rpt)],
                            alo_hbm.at[pl.ds(s * rpt, rpt)])
            if want_deg:
                pltpu.sync_copy(deg_s.at[pl.ds(s * rpt, rpt)],
                                deg_hbm.at[pl.ds(s * rpt, rpt)])

        @pl.when(c == 1)
        def _():
            pltpu.sync_copy(acc_s.at[pl.ds(s * rpt, rpt)],
                            ahi_hbm.at[pl.ds(s * rpt, rpt)])

    fn = pl.kernel(body, out_type=tuple(out_type), mesh=_MESH,
                   scratch_types=tuple(scratch))
    return fn(col2d, row2d, h_lo, h_hi)


def _sc_pair_gather_call(e0_2d, e1_2d, z):
    nw = NC * NS
    kbw = e0_2d.shape[0] // nw
    d_pad = e0_2d.shape[0] * 128

    out_type = (
        jax.ShapeDtypeStruct((d_pad, 128), jnp.float32),
        jax.ShapeDtypeStruct((d_pad, 128), jnp.float32),
    )
    scratch = (
        pltpu.VMEM((kbw, 128), jnp.int32),
        pltpu.VMEM((kbw, 128), jnp.int32),
        pltpu.VMEM((128, 128), jnp.float32),
        pltpu.VMEM((128, 128), jnp.float32),
        pltpu.SemaphoreType.DMA,
        pltpu.SemaphoreType.DMA,
    )

    def body(e0_hbm, e1_hbm, z_hbm, fi_hbm, fj_hbm,
             e0b, e1b, zi_v, zj_v, sem0, sem1):
        c = lax.axis_index("c")
        s = lax.axis_index("s")
        wid = s * NC + c
        blk0 = wid * kbw
        pltpu.sync_copy(e0_hbm.at[pl.ds(blk0, kbw)], e0b)
        pltpu.sync_copy(e1_hbm.at[pl.ds(blk0, kbw)], e1b)

        def eb(j, _):
            cp0 = pltpu.async_copy(z_hbm.at[e0b.at[j]], zi_v, sem0)
            cp1 = pltpu.async_copy(z_hbm.at[e1b.at[j]], zj_v, sem1)
            cp0.wait()
            pltpu.sync_copy(zi_v, fi_hbm.at[pl.ds((blk0 + j) * 128, 128)])
            cp1.wait()
            pltpu.sync_copy(zj_v, fj_hbm.at[pl.ds((blk0 + j) * 128, 128)])
            return 0

        lax.fori_loop(0, kbw, eb, 0)

    fn = pl.kernel(body, out_type=out_type, mesh=_MESH,
                   scratch_types=scratch)
    return fn(e0_2d, e1_2d, z)


def _tc_input_proj(x, w_in, b_in):
    n, _ = x.shape
    bn = 1000

    def body(x_ref, w_ref, b_ref, lo_ref, hi_ref):
        h = jnp.dot(x_ref[...], w_ref[...],
                    preferred_element_type=jnp.float32) + b_ref[...]
        lo_ref[...] = h[:, :128]
        hi_ref[...] = h[:, 128:]

    return pl.pallas_call(
        body,
        grid=(n // bn,),
        in_specs=[
            pl.BlockSpec((bn, x.shape[1]), lambda i: (i, 0)),
            pl.BlockSpec(w_in.shape, lambda i: (0, 0)),
            pl.BlockSpec(b_in.shape, lambda i: (0, 0)),
        ],
        out_specs=[
            pl.BlockSpec((bn, 128), lambda i: (i, 0)),
            pl.BlockSpec((bn, 128), lambda i: (i, 0)),
        ],
        out_shape=[
            jax.ShapeDtypeStruct((n, 128), jnp.float32),
            jax.ShapeDtypeStruct((n, 128), jnp.float32),
        ],
    )(x, w_in, b_in)


def _tc_conv_update(h_lo, h_hi, a_lo, a_hi, deg, w_c, b_c, g, be,
                    w_out=None, b_out=None):
    n = h_lo.shape[0]
    bn = 1000
    final = w_out is not None

    def body(hl, hh, al, ah, dg, wc, bc, gr, br, *rest):
        if final:
            wo, bo, z_ref = rest
        else:
            lo_ref, hi_ref = rest
        inv = 1.0 / jnp.maximum(dg[...], 1.0)
        alo = al[...] * inv
        ahi = ah[...] * inv
        t = (jnp.dot(alo, wc[:128, :], preferred_element_type=jnp.float32)
             + jnp.dot(ahi, wc[128:, :], preferred_element_type=jnp.float32)
             + bc[...])
        h = jnp.concatenate([hl[...], hh[...]], axis=1) + t
        m = jnp.mean(h, axis=1, keepdims=True)
        v = jnp.mean((h - m) ** 2, axis=1, keepdims=True)
        h = (h - m) * lax.rsqrt(v + 1e-5) * gr[...] + br[...]
        h = jnp.maximum(h, 0.0)
        if final:
            z_ref[...] = jnp.dot(h, wo[...],
                                 preferred_element_type=jnp.float32) + bo[...]
        else:
            lo_ref[...] = h[:, :128]
            hi_ref[...] = h[:, 128:]

    in_specs = [
        pl.BlockSpec((bn, 128), lambda i: (i, 0)),
        pl.BlockSpec((bn, 128), lambda i: (i, 0)),
        pl.BlockSpec((bn, 128), lambda i: (i, 0)),
        pl.BlockSpec((bn, 128), lambda i: (i, 0)),
        pl.BlockSpec((bn, 1), lambda i: (i, 0)),
        pl.BlockSpec(w_c.shape, lambda i: (0, 0)),
        pl.BlockSpec(b_c.shape, lambda i: (0, 0)),
        pl.BlockSpec(g.shape, lambda i: (0, 0)),
        pl.BlockSpec(be.shape, lambda i: (0, 0)),
    ]
    args = [h_lo, h_hi, a_lo, a_hi, deg, w_c, b_c, g, be]
    if final:
        in_specs += [
            pl.BlockSpec(w_out.shape, lambda i: (0, 0)),
            pl.BlockSpec(b_out.shape, lambda i: (0, 0)),
        ]
        args += [w_out, b_out]
        out_specs = pl.BlockSpec((bn, 128), lambda i: (i, 0))
        out_shape = jax.ShapeDtypeStruct((n, 128), jnp.float32)
    else:
        out_specs = [
            pl.BlockSpec((bn, 128), lambda i: (i, 0)),
            pl.BlockSpec((bn, 128), lambda i: (i, 0)),
        ]
        out_shape = [
            jax.ShapeDtypeStruct((n, 128), jnp.float32),
            jax.ShapeDtypeStruct((n, 128), jnp.float32),
        ]

    return pl.pallas_call(
        body, grid=(n // bn,), in_specs=in_specs,
        out_specs=out_specs, out_shape=out_shape,
    )(*args)


def _tc_decoder(fi, fj, w1a, w1b, b1, w2, b2, w3, b3):
    e = fi.shape[0]
    be_blk = 2560

    def body(fi_ref, fj_ref, w1a_ref, w1b_ref, b1_ref, w2_ref, b2_ref,
             w3_ref, b3_ref, out_ref):
        h = (jnp.dot(fi_ref[...], w1a_ref[...],
                     preferred_element_type=jnp.float32)
             + jnp.dot(fj_ref[...], w1b_ref[...],
                       preferred_element_type=jnp.float32)
             + b1_ref[...])
        h = jnp.maximum(h, 0.0)
        h = jnp.dot(h, w2_ref[...], preferred_element_type=jnp.float32) \
            + b2_ref[...]
        h = jnp.maximum(h, 0.0)
        s = jnp.dot(h, w3_ref[...], preferred_element_type=jnp.float32) \
            + b3_ref[...]
        out_ref[...] = s.reshape(1, -1)

    return pl.pallas_call(
        body,
        grid=(e // be_blk,),
        in_specs=[
            pl.BlockSpec((be_blk, 128), lambda i: (i, 0)),
            pl.BlockSpec((be_blk, 128), lambda i: (i, 0)),
            pl.BlockSpec(w1a.shape, lambda i: (0, 0)),
            pl.BlockSpec(w1b.shape, lambda i: (0, 0)),
            pl.BlockSpec(b1.shape, lambda i: (0, 0)),
            pl.BlockSpec(w2.shape, lambda i: (0, 0)),
            pl.BlockSpec(b2.shape, lambda i: (0, 0)),
            pl.BlockSpec(w3.shape, lambda i: (0, 0)),
            pl.BlockSpec(b3.shape, lambda i: (0, 0)),
        ],
        out_specs=pl.BlockSpec((1, be_blk), lambda i: (0, i)),
        out_shape=jax.ShapeDtypeStruct((1, e), jnp.float32),
    )(fi, fj, w1a, w1b, b1, w2, b2, w3, b3)


def _pad_idx_2d(idx, per_worker_blocks, workers, fill):
    total = workers * per_worker_blocks * 128
    pad = total - idx.shape[0]
    idx = jnp.concatenate(
        [idx, jnp.full((pad,), fill, jnp.int32)]) if pad else idx
    return idx.reshape(-1, 128)


def kernel(x, edge_index, edge_label_index, W_in, b_in, W_c0, b_c0, W_c1,
           b_c1, g0, be0, g1, be1, W_out, b_out, W_m1, b_m1, W_m2, b_m2,
           W_m3, b_m3):
    n = x.shape[0]
    e = edge_index.shape[1]
    n_pad = ((n + NS * 128 - 1) // (NS * 128)) * NS * 128

    row = edge_index[0]
    col = edge_index[1]
    kbt = -(-((e + NS * 128 - 1) // (NS * 128)) // 8) * 8
    col2d = _pad_idx_2d(col, kbt, NS, 0)
    row2d = _pad_idx_2d(row, kbt, NS, n_pad - 1)

    e0 = edge_label_index[0]
    e1 = edge_label_index[1]
    kbw = -(-((e + NC * NS * 128 - 1) // (NC * NS * 128)) // 8) * 8
    e0_2d = _pad_idx_2d(e0, kbw, NC * NS, 0)
    e1_2d = _pad_idx_2d(e1, kbw, NC * NS, 0)

    b_in2 = b_in.reshape(1, -1)
    h_lo, h_hi = _tc_input_proj(x, W_in, b_in2)

    a_lo, a_hi, deg = _sc_agg_call(col2d, row2d, h_lo, h_hi, n_pad, True)
    deg_n = deg[:n].reshape(n, 1)
    h_lo, h_hi = _tc_conv_update(
        h_lo, h_hi, a_lo[:n], a_hi[:n], deg_n, W_c0, b_c0.reshape(1, -1),
        g0.reshape(1, -1), be0.reshape(1, -1))

    a_lo, a_hi = _sc_agg_call(col2d, row2d, h_lo, h_hi, n_pad, False)
    z = _tc_conv_update(
        h_lo, h_hi, a_lo[:n], a_hi[:n], deg_n, W_c1, b_c1.reshape(1, -1),
        g1.reshape(1, -1), be1.reshape(1, -1),
        w_out=W_out, b_out=b_out.reshape(1, -1))

    fi, fj = _sc_pair_gather_call(e0_2d, e1_2d, z)

    scores2d = _tc_decoder(
        fi[:e], fj[:e], W_m1[:128], W_m1[128:], b_m1.reshape(1, -1),
        W_m2, b_m2.reshape(1, -1), W_m3, b_m3.reshape(1, -1))
    return scores2d[0]


# speedup vs baseline: 2.1752x; 1.1456x over previous
import functools

import jax
import jax.numpy as jnp
from jax import lax
from jax.experimental import pallas as pl
from jax.experimental.pallas import tpu as pltpu
from jax.experimental.pallas import tpu_sc as plsc

NC = 2
NS = 16
LANES = 16

_MESH = plsc.VectorSubcoreMesh(
    core_axis_name="c", subcore_axis_name="s", num_cores=NC, num_subcores=NS)


def _zero_vmem(buf, rows):
    z = jnp.zeros((LANES,), jnp.float32)

    def body(i, _):
        r = i // 8
        k = i % 8
        buf[r, pl.ds(k * LANES, LANES)] = z
        return 0

    lax.fori_loop(0, rows * 8, body, 0)


def _sc_agg_call(col2d, row2d, h_lo, h_hi, n_pad, want_deg):
    kbt = col2d.shape[0] // NS
    rpt = n_pad // NS
    zrows = 8
    nbuf = 2
    chunk = 32
    assert kbt % chunk == 0 and chunk % nbuf == 0 and rpt % zrows == 0

    out_type = [
        jax.ShapeDtypeStruct((n_pad, 128), jnp.float32),
        jax.ShapeDtypeStruct((n_pad, 128), jnp.float32),
    ]
    if want_deg:
        out_type.append(jax.ShapeDtypeStruct((n_pad,), jnp.float32))

    scratch = [
        pltpu.VMEM((chunk, 128), jnp.int32),
        pltpu.VMEM((chunk, 128), jnp.int32),
        pltpu.VMEM((zrows, 128), jnp.float32),
        pltpu.VMEM((128,), jnp.float32),
        pltpu.VMEM_SHARED((n_pad, 128), jnp.float32),
        pltpu.VMEM_SHARED((n_pad,), jnp.float32),
    ]
    scratch += [pltpu.VMEM((128, 128), jnp.float32) for _ in range(nbuf)]
    scratch += [pltpu.SemaphoreType.DMA for _ in range(nbuf)]

    def body(col_hbm, row_hbm, hlo_hbm, hhi_hbm, *rest):
        if want_deg:
            (alo_hbm, ahi_hbm, deg_hbm, colb, rowb, zbuf, onesb,
             acc_s, deg_s) = rest[:9]
            rings = rest[9:]
        else:
            (alo_hbm, ahi_hbm, colb, rowb, zbuf, onesb,
             acc_s, deg_s) = rest[:8]
            rings = rest[8:]
        rows_v = rings[:nbuf]
        sems = rings[nbuf:]
        c = lax.axis_index("c")
        s = lax.axis_index("s")

        _zero_vmem(zbuf, zrows)
        one = jnp.ones((LANES,), jnp.float32)
        for k in range(8):
            onesb[pl.ds(k * LANES, LANES)] = one

        def zc(k, _):
            pltpu.sync_copy(zbuf, acc_s.at[pl.ds(s * rpt + k * zrows, zrows)])
            return 0
        lax.fori_loop(0, rpt // zrows, zc, 0)

        def zd(k, _):
            pltpu.sync_copy(zbuf.at[0], deg_s.at[pl.ds(s * rpt + k * 128, 128)])
            return 0
        lax.fori_loop(0, rpt // 128, zd, 0)
        plsc.subcore_barrier()

        def run(h_hbm, with_deg):
            def cb(q, _):
                blk0 = s * kbt + q * chunk
                pltpu.sync_copy(col_hbm.at[pl.ds(blk0, chunk)], colb)
                pltpu.sync_copy(row_hbm.at[pl.ds(blk0, chunk)], rowb)

                for b in range(nbuf):
                    pltpu.async_copy(h_hbm.at[colb.at[b]], rows_v[b], sems[b])

                def step(tt, _):
                    for b in range(nbuf):
                        j = tt * nbuf + b
                        pltpu.make_async_copy(
                            h_hbm.at[colb.at[0]], rows_v[b], sems[b]).wait()
                        pltpu.sync_copy(rows_v[b], acc_s.at[rowb.at[j]],
                                        add=True)
                        if with_deg:
                            pltpu.sync_copy(onesb, deg_s.at[rowb.at[j]],
                                            add=True)

                        @pl.when(j + nbuf < chunk)
                        def _():
                            pltpu.async_copy(
                                h_hbm.at[colb.at[j + nbuf]], rows_v[b],
                                sems[b])
                    return 0

                lax.fori_loop(0, chunk // nbuf, step, 0)
                return 0

            lax.fori_loop(0, kbt // chunk, cb, 0)

        @pl.when(c == 0)
        def _():
            run(hlo_hbm, want_deg)

        @pl.when(c == 1)
        def _():
            run(hhi_hbm, False)

        plsc.subcore_barrier()

        @pl.when(c == 0)
        def _():
            pltpu.sync_copy(acc_s.at[pl.ds(s * rpt, rpt)],
                            alo_hbm.at[pl.ds(s * rpt, rpt)])
            if want_deg:
                pltpu.sync_copy(deg_s.at[pl.ds(s * rpt, rpt)],
                                deg_hbm.at[pl.ds(s * rpt, rpt)])

        @pl.when(c == 1)
        def _():
            pltpu.sync_copy(acc_s.at[pl.ds(s * rpt, rpt)],
                            ahi_hbm.at[pl.ds(s * rpt, rpt)])

    fn = pl.kernel(body, out_type=tuple(out_type), mesh=_MESH,
                   scratch_types=tuple(scratch))
    return fn(col2d, row2d, h_lo, h_hi)


def _sc_pair_gather_call(e0_2d, e1_2d, z):
    nw = NC * NS
    kbw = e0_2d.shape[0] // nw
    d_pad = e0_2d.shape[0] * 128

    nbuf = 2
    assert kbw % nbuf == 0

    out_type = (
        jax.ShapeDtypeStruct((d_pad, 128), jnp.float32),
        jax.ShapeDtypeStruct((d_pad, 128), jnp.float32),
    )
    scratch = (
        pltpu.VMEM((kbw, 128), jnp.int32),
        pltpu.VMEM((kbw, 128), jnp.int32),
        pltpu.VMEM((128, 128), jnp.float32),
        pltpu.VMEM((128, 128), jnp.float32),
        pltpu.VMEM((128, 128), jnp.float32),
        pltpu.VMEM((128, 128), jnp.float32),
        pltpu.SemaphoreType.DMA,
        pltpu.SemaphoreType.DMA,
        pltpu.SemaphoreType.DMA,
        pltpu.SemaphoreType.DMA,
        pltpu.SemaphoreType.DMA,
        pltpu.SemaphoreType.DMA,
        pltpu.SemaphoreType.DMA,
        pltpu.SemaphoreType.DMA,
    )

    def body(e0_hbm, e1_hbm, z_hbm, fi_hbm, fj_hbm,
             e0b, e1b, zi0, zi1, zj0, zj1,
             gsi0, gsi1, gsj0, gsj1, wsi0, wsi1, wsj0, wsj1):
        zi = (zi0, zi1)
        zj = (zj0, zj1)
        gsi = (gsi0, gsi1)
        gsj = (gsj0, gsj1)
        wsi = (wsi0, wsi1)
        wsj = (wsj0, wsj1)
        c = lax.axis_index("c")
        s = lax.axis_index("s")
        wid = s * NC + c
        blk0 = wid * kbw
        pltpu.sync_copy(e0_hbm.at[pl.ds(blk0, kbw)], e0b)
        pltpu.sync_copy(e1_hbm.at[pl.ds(blk0, kbw)], e1b)

        for b in range(nbuf):
            pltpu.async_copy(z_hbm.at[e0b.at[b]], zi[b], gsi[b])
            pltpu.async_copy(z_hbm.at[e1b.at[b]], zj[b], gsj[b])

        def step(tt, _):
            for b in range(nbuf):
                j = tt * nbuf + b
                pltpu.make_async_copy(
                    z_hbm.at[e0b.at[0]], zi[b], gsi[b]).wait()
                pltpu.async_copy(
                    zi[b], fi_hbm.at[pl.ds((blk0 + j) * 128, 128)], wsi[b])
                pltpu.make_async_copy(
                    z_hbm.at[e1b.at[0]], zj[b], gsj[b]).wait()
                pltpu.async_copy(
                    zj[b], fj_hbm.at[pl.ds((blk0 + j) * 128, 128)], wsj[b])

                @pl.when(j + nbuf < kbw)
                def _():
                    pltpu.make_async_copy(
                        zi[b], fi_hbm.at[pl.ds(0, 128)], wsi[b]).wait()
                    pltpu.async_copy(z_hbm.at[e0b.at[j + nbuf]], zi[b], gsi[b])
                    pltpu.make_async_copy(
                        zj[b], fj_hbm.at[pl.ds(0, 128)], wsj[b]).wait()
                    pltpu.async_copy(z_hbm.at[e1b.at[j + nbuf]], zj[b], gsj[b])
            return 0

        lax.fori_loop(0, kbw // nbuf, step, 0)

        for b in range(nbuf):
            pltpu.make_async_copy(
                zi[b], fi_hbm.at[pl.ds(0, 128)], wsi[b]).wait()
            pltpu.make_async_copy(
                zj[b], fj_hbm.at[pl.ds(0, 128)], wsj[b]).wait()

    fn = pl.kernel(body, out_type=out_type, mesh=_MESH,
                   scratch_types=scratch)
    return fn(e0_2d, e1_2d, z)


def _tc_input_proj(x, w_in, b_in):
    n, _ = x.shape
    bn = 1000

    def body(x_ref, w_ref, b_ref, lo_ref, hi_ref):
        h = jnp.dot(x_ref[...], w_ref[...],
                    preferred_element_type=jnp.float32) + b_ref[...]
        lo_ref[...] = h[:, :128]
        hi_ref[...] = h[:, 128:]

    return pl.pallas_call(
        body,
        grid=(n // bn,),
        in_specs=[
            pl.BlockSpec((bn, x.shape[1]), lambda i: (i, 0)),
            pl.BlockSpec(w_in.shape, lambda i: (0, 0)),
            pl.BlockSpec(b_in.shape, lambda i: (0, 0)),
        ],
        out_specs=[
            pl.BlockSpec((bn, 128), lambda i: (i, 0)),
            pl.BlockSpec((bn, 128), lambda i: (i, 0)),
        ],
        out_shape=[
            jax.ShapeDtypeStruct((n, 128), jnp.float32),
            jax.ShapeDtypeStruct((n, 128), jnp.float32),
        ],
    )(x, w_in, b_in)


def _tc_conv_update(h_lo, h_hi, a_lo, a_hi, deg, w_c, b_c, g, be,
                    w_out=None, b_out=None):
    n = h_lo.shape[0]
    bn = 1000
    final = w_out is not None

    def body(hl, hh, al, ah, dg, wc, bc, gr, br, *rest):
        if final:
            wo, bo, z_ref = rest
        else:
            lo_ref, hi_ref = rest
        inv = 1.0 / jnp.maximum(dg[...], 1.0)
        alo = al[...] * inv
        ahi = ah[...] * inv
        t = (jnp.dot(alo, wc[:128, :], preferred_element_type=jnp.float32)
             + jnp.dot(ahi, wc[128:, :], preferred_element_type=jnp.float32)
             + bc[...])
        h = jnp.concatenate([hl[...], hh[...]], axis=1) + t
        m = jnp.mean(h, axis=1, keepdims=True)
        v = jnp.mean((h - m) ** 2, axis=1, keepdims=True)
        h = (h - m) * lax.rsqrt(v + 1e-5) * gr[...] + br[...]
        h = jnp.maximum(h, 0.0)
        if final:
            z_ref[...] = jnp.dot(h, wo[...],
                                 preferred_element_type=jnp.float32) + bo[...]
        else:
            lo_ref[...] = h[:, :128]
            hi_ref[...] = h[:, 128:]

    in_specs = [
        pl.BlockSpec((bn, 128), lambda i: (i, 0)),
        pl.BlockSpec((bn, 128), lambda i: (i, 0)),
        pl.BlockSpec((bn, 128), lambda i: (i, 0)),
        pl.BlockSpec((bn, 128), lambda i: (i, 0)),
        pl.BlockSpec((bn, 1), lambda i: (i, 0)),
        pl.BlockSpec(w_c.shape, lambda i: (0, 0)),
        pl.BlockSpec(b_c.shape, lambda i: (0, 0)),
        pl.BlockSpec(g.shape, lambda i: (0, 0)),
        pl.BlockSpec(be.shape, lambda i: (0, 0)),
    ]
    args = [h_lo, h_hi, a_lo, a_hi, deg, w_c, b_c, g, be]
    if final:
        in_specs += [
            pl.BlockSpec(w_out.shape, lambda i: (0, 0)),
            pl.BlockSpec(b_out.shape, lambda i: (0, 0)),
        ]
        args += [w_out, b_out]
        out_specs = pl.BlockSpec((bn, 128), lambda i: (i, 0))
        out_shape = jax.ShapeDtypeStruct((n, 128), jnp.float32)
    else:
        out_specs = [
            pl.BlockSpec((bn, 128), lambda i: (i, 0)),
            pl.BlockSpec((bn, 128), lambda i: (i, 0)),
        ]
        out_shape = [
            jax.ShapeDtypeStruct((n, 128), jnp.float32),
            jax.ShapeDtypeStruct((n, 128), jnp.float32),
        ]

    return pl.pallas_call(
        body, grid=(n // bn,), in_specs=in_specs,
        out_specs=out_specs, out_shape=out_shape,
    )(*args)


def _tc_decoder(fi, fj, w1a, w1b, b1, w2, b2, w3, b3):
    e = fi.shape[0]
    be_blk = 2560

    def body(fi_ref, fj_ref, w1a_ref, w1b_ref, b1_ref, w2_ref, b2_ref,
             w3_ref, b3_ref, out_ref):
        h = (jnp.dot(fi_ref[...], w1a_ref[...],
                     preferred_element_type=jnp.float32)
             + jnp.dot(fj_ref[...], w1b_ref[...],
                       preferred_element_type=jnp.float32)
             + b1_ref[...])
        h = jnp.maximum(h, 0.0)
        h = jnp.dot(h, w2_ref[...], preferred_element_type=jnp.float32) \
            + b2_ref[...]
        h = jnp.maximum(h, 0.0)
        s = jnp.dot(h, w3_ref[...], preferred_element_type=jnp.float32) \
            + b3_ref[...]
        out_ref[...] = s.reshape(1, -1)

    return pl.pallas_call(
        body,
        grid=(e // be_blk,),
        in_specs=[
            pl.BlockSpec((be_blk, 128), lambda i: (i, 0)),
            pl.BlockSpec((be_blk, 128), lambda i: (i, 0)),
            pl.BlockSpec(w1a.shape, lambda i: (0, 0)),
            pl.BlockSpec(w1b.shape, lambda i: (0, 0)),
            pl.BlockSpec(b1.shape, lambda i: (0, 0)),
            pl.BlockSpec(w2.shape, lambda i: (0, 0)),
            pl.BlockSpec(b2.shape, lambda i: (0, 0)),
            pl.BlockSpec(w3.shape, lambda i: (0, 0)),
            pl.BlockSpec(b3.shape, lambda i: (0, 0)),
        ],
        out_specs=pl.BlockSpec((1, be_blk), lambda i: (0, i)),
        out_shape=jax.ShapeDtypeStruct((1, e), jnp.float32),
    )(fi, fj, w1a, w1b, b1, w2, b2, w3, b3)


def _pad_idx_2d(idx, per_worker_blocks, workers, fill):
    total = workers * per_worker_blocks * 128
    pad = total - idx.shape[0]
    idx = jnp.concatenate(
        [idx, jnp.full((pad,), fill, jnp.int32)]) if pad else idx
    return idx.reshape(-1, 128)


def kernel(x, edge_index, edge_label_index, W_in, b_in, W_c0, b_c0, W_c1,
           b_c1, g0, be0, g1, be1, W_out, b_out, W_m1, b_m1, W_m2, b_m2,
           W_m3, b_m3):
    n = x.shape[0]
    e = edge_index.shape[1]
    n_pad = ((n + NS * 128 - 1) // (NS * 128)) * NS * 128

    row = edge_index[0]
    col = edge_index[1]
    kbt = -(-((e + NS * 128 - 1) // (NS * 128)) // 8) * 8
    col2d = _pad_idx_2d(col, kbt, NS, 0)
    row2d = _pad_idx_2d(row, kbt, NS, n_pad - 1)

    e0 = edge_label_index[0]
    e1 = edge_label_index[1]
    kbw = -(-((e + NC * NS * 128 - 1) // (NC * NS * 128)) // 8) * 8
    e0_2d = _pad_idx_2d(e0, kbw, NC * NS, 0)
    e1_2d = _pad_idx_2d(e1, kbw, NC * NS, 0)

    b_in2 = b_in.reshape(1, -1)
    h_lo, h_hi = _tc_input_proj(x, W_in, b_in2)

    a_lo, a_hi, deg = _sc_agg_call(col2d, row2d, h_lo, h_hi, n_pad, True)
    deg_n = deg[:n].reshape(n, 1)
    h_lo, h_hi = _tc_conv_update(
        h_lo, h_hi, a_lo[:n], a_hi[:n], deg_n, W_c0, b_c0.reshape(1, -1),
        g0.reshape(1, -1), be0.reshape(1, -1))

    a_lo, a_hi = _sc_agg_call(col2d, row2d, h_lo, h_hi, n_pad, False)
    z = _tc_conv_update(
        h_lo, h_hi, a_lo[:n], a_hi[:n], deg_n, W_c1, b_c1.reshape(1, -1),
        g1.reshape(1, -1), be1.reshape(1, -1),
        w_out=W_out, b_out=b_out.reshape(1, -1))

    fi, fj = _sc_pair_gather_call(e0_2d, e1_2d, z)

    scores2d = _tc_decoder(
        fi[:e], fj[:e], W_m1[:128], W_m1[128:], b_m1.reshape(1, -1),
        W_m2, b_m2.reshape(1, -1), W_m3, b_m3.reshape(1, -1))
    return scores2d[0]


# speedup vs baseline: 2.4012x; 1.1039x over previous
import functools

import jax
import jax.numpy as jnp
from jax import lax
from jax.experimental import pallas as pl
from jax.experimental.pallas import tpu as pltpu
from jax.experimental.pallas import tpu_sc as plsc

NC = 2
NS = 16
LANES = 16

_MESH = plsc.VectorSubcoreMesh(
    core_axis_name="c", subcore_axis_name="s", num_cores=NC, num_subcores=NS)


def _zero_vmem(buf, rows):
    z = jnp.zeros((LANES,), jnp.float32)

    def body(i, _):
        r = i // 8
        k = i % 8
        buf[r, pl.ds(k * LANES, LANES)] = z
        return 0

    lax.fori_loop(0, rows * 8, body, 0)


def _sc_agg_call(col2d, row2d, h_lo, h_hi, n_pad, want_deg):
    kbt = col2d.shape[0] // NS
    rpt = n_pad // NS
    zrows = 8
    nbuf = 2
    chunk = 32
    assert kbt % chunk == 0 and chunk % nbuf == 0 and rpt % zrows == 0

    out_type = [
        jax.ShapeDtypeStruct((n_pad, 128), jnp.float32),
        jax.ShapeDtypeStruct((n_pad, 128), jnp.float32),
    ]
    if want_deg:
        out_type.append(jax.ShapeDtypeStruct((n_pad,), jnp.float32))

    scratch = [
        pltpu.VMEM((chunk, 128), jnp.int32),
        pltpu.VMEM((chunk, 128), jnp.int32),
        pltpu.VMEM((zrows, 128), jnp.float32),
        pltpu.VMEM((128,), jnp.float32),
        pltpu.VMEM_SHARED((n_pad, 128), jnp.float32),
        pltpu.VMEM_SHARED((n_pad,), jnp.float32),
    ]
    scratch += [pltpu.VMEM((128, 128), jnp.float32) for _ in range(nbuf)]
    scratch += [pltpu.SemaphoreType.DMA for _ in range(nbuf)]

    def body(col_hbm, row_hbm, hlo_hbm, hhi_hbm, *rest):
        if want_deg:
            (alo_hbm, ahi_hbm, deg_hbm, colb, rowb, zbuf, onesb,
             acc_s, deg_s) = rest[:9]
            rings = rest[9:]
        else:
            (alo_hbm, ahi_hbm, colb, rowb, zbuf, onesb,
             acc_s, deg_s) = rest[:8]
            rings = rest[8:]
        rows_v = rings[:nbuf]
        sems = rings[nbuf:]
        c = lax.axis_index("c")
        s = lax.axis_index("s")

        _zero_vmem(zbuf, zrows)
        one = jnp.ones((LANES,), jnp.float32)
        for k in range(8):
            onesb[pl.ds(k * LANES, LANES)] = one

        def zc(k, _):
            pltpu.sync_copy(zbuf, acc_s.at[pl.ds(s * rpt + k * zrows, zrows)])
            return 0
        lax.fori_loop(0, rpt // zrows, zc, 0)

        def zd(k, _):
            pltpu.sync_copy(zbuf.at[0], deg_s.at[pl.ds(s * rpt + k * 128, 128)])
            return 0
        lax.fori_loop(0, rpt // 128, zd, 0)
        plsc.subcore_barrier()

        def run(h_hbm, with_deg):
            def cb(q, _):
                blk0 = s * kbt + q * chunk
                pltpu.sync_copy(col_hbm.at[pl.ds(blk0, chunk)], colb)
                pltpu.sync_copy(row_hbm.at[pl.ds(blk0, chunk)], rowb)

                for b in range(nbuf):
                    pltpu.async_copy(h_hbm.at[colb.at[b]], rows_v[b], sems[b])

                def step(tt, _):
                    for b in range(nbuf):
                        j = tt * nbuf + b
                        pltpu.make_async_copy(
                            h_hbm.at[colb.at[0]], rows_v[b], sems[b]).wait()
                        pltpu.sync_copy(rows_v[b], acc_s.at[rowb.at[j]],
                                        add=True)
                        if with_deg:
                            pltpu.sync_copy(onesb, deg_s.at[rowb.at[j]],
                                            add=True)

                        @pl.when(j + nbuf < chunk)
                        def _():
                            pltpu.async_copy(
                                h_hbm.at[colb.at[j + nbuf]], rows_v[b],
                                sems[b])
                    return 0

                lax.fori_loop(0, chunk // nbuf, step, 0)
                return 0

            lax.fori_loop(0, kbt // chunk, cb, 0)

        @pl.when(c == 0)
        def _():
            run(hlo_hbm, want_deg)

        @pl.when(c == 1)
        def _():
            run(hhi_hbm, False)

        plsc.subcore_barrier()

        @pl.when(c == 0)
        def _():
            pltpu.sync_copy(acc_s.at[pl.ds(s * rpt, rpt)],
                            alo_hbm.at[pl.ds(s * rpt, rpt)])
            if want_deg:
                pltpu.sync_copy(deg_s.at[pl.ds(s * rpt, rpt)],
                                deg_hbm.at[pl.ds(s * rpt, rpt)])

        @pl.when(c == 1)
        def _():
            pltpu.sync_copy(acc_s.at[pl.ds(s * rpt, rpt)],
                            ahi_hbm.at[pl.ds(s * rpt, rpt)])

    fn = pl.kernel(body, out_type=tuple(out_type), mesh=_MESH,
                   scratch_types=tuple(scratch))
    return fn(col2d, row2d, h_lo, h_hi)


def _sc_pair_gather_call(e0_2d, e1_2d, za, zb):
    nw = NC * NS
    kbw = e0_2d.shape[0] // nw
    d_pad = e0_2d.shape[0] * 128
    dt = za.dtype

    nbuf = 2
    assert kbw % nbuf == 0

    out_type = (
        jax.ShapeDtypeStruct((d_pad, 128), dt),
        jax.ShapeDtypeStruct((d_pad, 128), dt),
    )
    scratch = (
        pltpu.VMEM((kbw, 128), jnp.int32),
        pltpu.VMEM((kbw, 128), jnp.int32),
        pltpu.VMEM((128, 128), dt),
        pltpu.VMEM((128, 128), dt),
        pltpu.VMEM((128, 128), dt),
        pltpu.VMEM((128, 128), dt),
        pltpu.SemaphoreType.DMA,
        pltpu.SemaphoreType.DMA,
        pltpu.SemaphoreType.DMA,
        pltpu.SemaphoreType.DMA,
        pltpu.SemaphoreType.DMA,
        pltpu.SemaphoreType.DMA,
        pltpu.SemaphoreType.DMA,
        pltpu.SemaphoreType.DMA,
    )

    def body(e0_hbm, e1_hbm, za_hbm, zb_hbm, fi_hbm, fj_hbm,
             e0b, e1b, zi0, zi1, zj0, zj1,
             gsi0, gsi1, gsj0, gsj1, wsi0, wsi1, wsj0, wsj1):
        zi = (zi0, zi1)
        zj = (zj0, zj1)
        gsi = (gsi0, gsi1)
        gsj = (gsj0, gsj1)
        wsi = (wsi0, wsi1)
        wsj = (wsj0, wsj1)
        c = lax.axis_index("c")
        s = lax.axis_index("s")
        wid = s * NC + c
        blk0 = wid * kbw
        pltpu.sync_copy(e0_hbm.at[pl.ds(blk0, kbw)], e0b)
        pltpu.sync_copy(e1_hbm.at[pl.ds(blk0, kbw)], e1b)

        for b in range(nbuf):
            pltpu.async_copy(za_hbm.at[e0b.at[b]], zi[b], gsi[b])
            pltpu.async_copy(zb_hbm.at[e1b.at[b]], zj[b], gsj[b])

        def step(tt, _):
            for b in range(nbuf):
                j = tt * nbuf + b
                pltpu.make_async_copy(
                    za_hbm.at[e0b.at[0]], zi[b], gsi[b]).wait()
                pltpu.async_copy(
                    zi[b], fi_hbm.at[pl.ds((blk0 + j) * 128, 128)], wsi[b])
                pltpu.make_async_copy(
                    zb_hbm.at[e1b.at[0]], zj[b], gsj[b]).wait()
                pltpu.async_copy(
                    zj[b], fj_hbm.at[pl.ds((blk0 + j) * 128, 128)], wsj[b])

                @pl.when(j + nbuf < kbw)
                def _():
                    pltpu.make_async_copy(
                        zi[b], fi_hbm.at[pl.ds(0, 128)], wsi[b]).wait()
                    pltpu.async_copy(za_hbm.at[e0b.at[j + nbuf]], zi[b], gsi[b])
                    pltpu.make_async_copy(
                        zj[b], fj_hbm.at[pl.ds(0, 128)], wsj[b]).wait()
                    pltpu.async_copy(zb_hbm.at[e1b.at[j + nbuf]], zj[b], gsj[b])
            return 0

        lax.fori_loop(0, kbw // nbuf, step, 0)

        for b in range(nbuf):
            pltpu.make_async_copy(
                zi[b], fi_hbm.at[pl.ds(0, 128)], wsi[b]).wait()
            pltpu.make_async_copy(
                zj[b], fj_hbm.at[pl.ds(0, 128)], wsj[b]).wait()

    fn = pl.kernel(body, out_type=out_type, mesh=_MESH,
                   scratch_types=scratch)
    return fn(e0_2d, e1_2d, za, zb)


def _tc_input_proj(x, w_in, b_in):
    n, _ = x.shape
    bn = 1000

    def body(x_ref, w_ref, b_ref, lo_ref, hi_ref):
        h = jnp.dot(x_ref[...], w_ref[...],
                    preferred_element_type=jnp.float32) + b_ref[...]
        lo_ref[...] = h[:, :128]
        hi_ref[...] = h[:, 128:]

    return pl.pallas_call(
        body,
        grid=(n // bn,),
        in_specs=[
            pl.BlockSpec((bn, x.shape[1]), lambda i: (i, 0)),
            pl.BlockSpec(w_in.shape, lambda i: (0, 0)),
            pl.BlockSpec(b_in.shape, lambda i: (0, 0)),
        ],
        out_specs=[
            pl.BlockSpec((bn, 128), lambda i: (i, 0)),
            pl.BlockSpec((bn, 128), lambda i: (i, 0)),
        ],
        out_shape=[
            jax.ShapeDtypeStruct((n, 128), jnp.float32),
            jax.ShapeDtypeStruct((n, 128), jnp.float32),
        ],
    )(x, w_in, b_in)


def _pack_bf16(t):
    lo = t[:, :128].astype(jnp.bfloat16).astype(jnp.float32)
    hi = t[:, 128:].astype(jnp.bfloat16).astype(jnp.float32)
    rl = lax.bitcast_convert_type(lo, jnp.uint32)
    rh = lax.bitcast_convert_type(hi, jnp.uint32)
    return lax.bitcast_convert_type((rl >> 16) | rh, jnp.int32)


def _tc_conv_update(h_lo, h_hi, a_lo, a_hi, deg, w_c, b_c, g, be,
                    w_a=None, b_a=None, w_b=None, b_b=None):
    n = h_lo.shape[0]
    bn = 1000
    final = w_a is not None

    def body(hl, hh, al, ah, dg, wc, bc, gr, br, *rest):
        if final:
            wa, ba, wb, bb, apk_ref, bpk_ref = rest
        else:
            lo_ref, hi_ref = rest
        inv = 1.0 / jnp.maximum(dg[...], 1.0)
        alo = al[...] * inv
        ahi = ah[...] * inv
        t = (jnp.dot(alo, wc[:128, :], preferred_element_type=jnp.float32)
             + jnp.dot(ahi, wc[128:, :], preferred_element_type=jnp.float32)
             + bc[...])
        h = jnp.concatenate([hl[...], hh[...]], axis=1) + t
        m = jnp.mean(h, axis=1, keepdims=True)
        v = jnp.mean((h - m) ** 2, axis=1, keepdims=True)
        h = (h - m) * lax.rsqrt(v + 1e-5) * gr[...] + br[...]
        h = jnp.maximum(h, 0.0)
        if final:
            a = jnp.dot(h, wa[...],
                        preferred_element_type=jnp.float32) + ba[...]
            b = jnp.dot(h, wb[...],
                        preferred_element_type=jnp.float32) + bb[...]
            apk_ref[...] = _pack_bf16(a)
            bpk_ref[...] = _pack_bf16(b)
        else:
            lo_ref[...] = h[:, :128]
            hi_ref[...] = h[:, 128:]

    in_specs = [
        pl.BlockSpec((bn, 128), lambda i: (i, 0)),
        pl.BlockSpec((bn, 128), lambda i: (i, 0)),
        pl.BlockSpec((bn, 128), lambda i: (i, 0)),
        pl.BlockSpec((bn, 128), lambda i: (i, 0)),
        pl.BlockSpec((bn, 1), lambda i: (i, 0)),
        pl.BlockSpec(w_c.shape, lambda i: (0, 0)),
        pl.BlockSpec(b_c.shape, lambda i: (0, 0)),
        pl.BlockSpec(g.shape, lambda i: (0, 0)),
        pl.BlockSpec(be.shape, lambda i: (0, 0)),
    ]
    args = [h_lo, h_hi, a_lo, a_hi, deg, w_c, b_c, g, be]
    if final:
        in_specs += [
            pl.BlockSpec(w_a.shape, lambda i: (0, 0)),
            pl.BlockSpec(b_a.shape, lambda i: (0, 0)),
            pl.BlockSpec(w_b.shape, lambda i: (0, 0)),
            pl.BlockSpec(b_b.shape, lambda i: (0, 0)),
        ]
        args += [w_a, b_a, w_b, b_b]
        out_specs = [
            pl.BlockSpec((bn, 128), lambda i: (i, 0)),
            pl.BlockSpec((bn, 128), lambda i: (i, 0)),
        ]
        out_shape = [
            jax.ShapeDtypeStruct((n, 128), jnp.int32),
            jax.ShapeDtypeStruct((n, 128), jnp.int32),
        ]
    else:
        out_specs = [
            pl.BlockSpec((bn, 128), lambda i: (i, 0)),
            pl.BlockSpec((bn, 128), lambda i: (i, 0)),
        ]
        out_shape = [
            jax.ShapeDtypeStruct((n, 128), jnp.float32),
            jax.ShapeDtypeStruct((n, 128), jnp.float32),
        ]

    return pl.pallas_call(
        body, grid=(n // bn,), in_specs=in_specs,
        out_specs=out_specs, out_shape=out_shape,
    )(*args)


def _unpack_bf16(v):
    vu = lax.bitcast_convert_type(v, jnp.uint32)
    lo = lax.bitcast_convert_type(vu << 16, jnp.float32)
    hi = lax.bitcast_convert_type(vu & jnp.uint32(0xFFFF0000), jnp.float32)
    return lo, hi


def _tc_decoder(fi, fj, b1a, b1b, w2a, w2b, b2, w3, b3):
    e = fi.shape[0]
    be_blk = 2560

    def body(fi_ref, fj_ref, b1a_ref, b1b_ref, w2a_ref, w2b_ref, b2_ref,
             w3_ref, b3_ref, out_ref):
        alo, ahi = _unpack_bf16(fi_ref[...])
        blo, bhi = _unpack_bf16(fj_ref[...])
        hlo = jnp.maximum(alo + blo + b1a_ref[...], 0.0)
        hhi = jnp.maximum(ahi + bhi + b1b_ref[...], 0.0)
        h = (jnp.dot(hlo, w2a_ref[...], preferred_element_type=jnp.float32)
             + jnp.dot(hhi, w2b_ref[...], preferred_element_type=jnp.float32)
             + b2_ref[...])
        h = jnp.maximum(h, 0.0)
        s = jnp.dot(h, w3_ref[...], preferred_element_type=jnp.float32) \
            + b3_ref[...]
        out_ref[...] = s.reshape(1, -1)

    return pl.pallas_call(
        body,
        grid=(e // be_blk,),
        in_specs=[
            pl.BlockSpec((be_blk, 128), lambda i: (i, 0)),
            pl.BlockSpec((be_blk, 128), lambda i: (i, 0)),
            pl.BlockSpec(b1a.shape, lambda i: (0, 0)),
            pl.BlockSpec(b1b.shape, lambda i: (0, 0)),
            pl.BlockSpec(w2a.shape, lambda i: (0, 0)),
            pl.BlockSpec(w2b.shape, lambda i: (0, 0)),
            pl.BlockSpec(b2.shape, lambda i: (0, 0)),
            pl.BlockSpec(w3.shape, lambda i: (0, 0)),
            pl.BlockSpec(b3.shape, lambda i: (0, 0)),
        ],
        out_specs=pl.BlockSpec((1, be_blk), lambda i: (0, i)),
        out_shape=jax.ShapeDtypeStruct((1, e), jnp.float32),
    )(fi, fj, b1a, b1b, w2a, w2b, b2, w3, b3)


def _pad_idx_2d(idx, per_worker_blocks, workers, fill):
    total = workers * per_worker_blocks * 128
    pad = total - idx.shape[0]
    idx = jnp.concatenate(
        [idx, jnp.full((pad,), fill, jnp.int32)]) if pad else idx
    return idx.reshape(-1, 128)


def kernel(x, edge_index, edge_label_index, W_in, b_in, W_c0, b_c0, W_c1,
           b_c1, g0, be0, g1, be1, W_out, b_out, W_m1, b_m1, W_m2, b_m2,
           W_m3, b_m3):
    n = x.shape[0]
    e = edge_index.shape[1]
    n_pad = ((n + NS * 128 - 1) // (NS * 128)) * NS * 128

    row = edge_index[0]
    col = edge_index[1]
    kbt = -(-((e + NS * 128 - 1) // (NS * 128)) // 8) * 8
    col2d = _pad_idx_2d(col, kbt, NS, 0)
    row2d = _pad_idx_2d(row, kbt, NS, n_pad - 1)

    e0 = edge_label_index[0]
    e1 = edge_label_index[1]
    kbw = -(-((e + NC * NS * 128 - 1) // (NC * NS * 128)) // 8) * 8
    e0_2d = _pad_idx_2d(e0, kbw, NC * NS, 0)
    e1_2d = _pad_idx_2d(e1, kbw, NC * NS, 0)

    b_in2 = b_in.reshape(1, -1)
    h_lo, h_hi = _tc_input_proj(x, W_in, b_in2)

    a_lo, a_hi, deg = _sc_agg_call(col2d, row2d, h_lo, h_hi, n_pad, True)
    deg_n = deg[:n].reshape(n, 1)
    h_lo, h_hi = _tc_conv_update(
        h_lo, h_hi, a_lo[:n], a_hi[:n], deg_n, W_c0, b_c0.reshape(1, -1),
        g0.reshape(1, -1), be0.reshape(1, -1))

    a_lo, a_hi = _sc_agg_call(col2d, row2d, h_lo, h_hi, n_pad, False)

    w_a = W_out @ W_m1[:128]
    b_a = b_out @ W_m1[:128]
    w_b = W_out @ W_m1[128:]
    b_b = b_out @ W_m1[128:]
    apk, bpk = _tc_conv_update(
        h_lo, h_hi, a_lo[:n], a_hi[:n], deg_n, W_c1, b_c1.reshape(1, -1),
        g1.reshape(1, -1), be1.reshape(1, -1),
        w_a=w_a, b_a=b_a.reshape(1, -1), w_b=w_b, b_b=b_b.reshape(1, -1))

    fi, fj = _sc_pair_gather_call(e0_2d, e1_2d, apk, bpk)

    scores2d = _tc_decoder(
        fi[:e], fj[:e], b_m1[:128].reshape(1, -1), b_m1[128:].reshape(1, -1),
        W_m2[:128], W_m2[128:], b_m2.reshape(1, -1), W_m3,
        b_m3.reshape(1, -1))
    return scores2d[0]
